# Initial kernel scaffold; baseline (speedup 1.0000x reference)
import jax
import jax.numpy as jnp
from jax.experimental import pallas as pl


def kernel(x0, x1, edge_index, Wp0, bp0, Wp1, bp1, W1, b1, W2, b2):
    raise NotImplementedError("write your pallas kernel here")


# speedup vs baseline: 10.2080x; 10.2080x over previous
import functools

import jax
import jax.numpy as jnp
from jax import lax
from jax.experimental import pallas as pl
from jax.experimental.pallas import tpu as pltpu
from jax.experimental.pallas import tpu_sc as plsc

NC = 2
NS = 16
NW = NC * NS
CHUNK = 128
DEG_W = 16
BM = 1000


def _sc_mesh():
    return plsc.VectorSubcoreMesh(core_axis_name="c", subcore_axis_name="s")


def _make_deg_kernel(n_pad, cpw):
    rps = n_pad // NS

    @functools.partial(
        pl.kernel,
        out_type=jax.ShapeDtypeStruct((NC, n_pad, DEG_W), jnp.float32),
        mesh=_sc_mesh(),
        compiler_params=pltpu.CompilerParams(use_tc_tiling_on_sc=False),
        scratch_types=[
            pltpu.VMEM((CHUNK,), jnp.int32),
            pltpu.VMEM((CHUNK, DEG_W), jnp.float32),
            pltpu.VMEM_SHARED((n_pad, DEG_W), jnp.float32),
        ],
    )
    def deg_kernel(dsts_hbm, ones_hbm, zeros_hbm, out_hbm, dst_v, ones_v, acc_sh):
        c = lax.axis_index("c")
        s = lax.axis_index("s")
        w = c * NS + s
        pltpu.sync_copy(zeros_hbm, acc_sh.at[pl.ds(s * rps, rps)])
        pltpu.sync_copy(ones_hbm, ones_v)
        plsc.subcore_barrier()

        def body(j, carry):
            row = w * cpw + j
            pltpu.sync_copy(dsts_hbm.at[row], dst_v)
            pltpu.sync_copy(ones_v, acc_sh.at[dst_v], add=True)
            return carry

        lax.fori_loop(0, cpw, body, 0)
        plsc.subcore_barrier()
        pltpu.sync_copy(acc_sh.at[pl.ds(s * rps, rps)],
                        out_hbm.at[c, pl.ds(s * rps, rps)])

    return deg_kernel


def _make_agg_kernel(d, n_pad, cpw):
    rps = n_pad // NS

    @functools.partial(
        pl.kernel,
        out_type=jax.ShapeDtypeStruct((NC, n_pad, d), jnp.float32),
        mesh=_sc_mesh(),
        scratch_types=[
            pltpu.VMEM((CHUNK,), jnp.int32),
            pltpu.VMEM((CHUNK,), jnp.int32),
            pltpu.VMEM((CHUNK, d), jnp.float32),
            pltpu.SemaphoreType.DMA,
            pltpu.VMEM_SHARED((n_pad, d), jnp.float32),
        ],
    )
    def agg_kernel(u_hbm, srcs_hbm, dsts_hbm, zeros_hbm, out_hbm,
                   src_v, dst_v, rows_v, sem, acc_sh):
        c = lax.axis_index("c")
        s = lax.axis_index("s")
        w = c * NS + s
        pltpu.sync_copy(zeros_hbm, acc_sh.at[pl.ds(s * rps, rps)])
        plsc.subcore_barrier()

        def body(j, carry):
            row = w * cpw + j
            pltpu.sync_copy(srcs_hbm.at[row], src_v)
            pltpu.sync_copy(dsts_hbm.at[row], dst_v)
            pltpu.async_copy(u_hbm.at[src_v], rows_v, sem).wait()
            pltpu.sync_copy(rows_v, acc_sh.at[dst_v], add=True)
            return carry

        lax.fori_loop(0, cpw, body, 0)
        plsc.subcore_barrier()
        pltpu.sync_copy(acc_sh.at[pl.ds(s * rps, rps)],
                        out_hbm.at[c, pl.ds(s * rps, rps)])

    return agg_kernel


def _proj_body(x_ref, w_ref, b_ref, w1_ref, degp_ref, u_ref, dinv_ref):
    x = x_ref[...]
    xb = jnp.dot(x, w_ref[0], preferred_element_type=jnp.float32,
                 precision=lax.Precision.HIGHEST) + b_ref[0]
    t1 = jnp.dot(xb, w1_ref[...], preferred_element_type=jnp.float32,
                 precision=lax.Precision.HIGHEST)
    deg = degp_ref[0, :, 0] + degp_ref[1, :, 0] + 1.0
    dinv = lax.rsqrt(deg)
    u_ref[...] = t1 * dinv[:, None]
    dinv_ref[...] = jnp.broadcast_to(dinv[:, None], dinv_ref.shape)


def _mid_body(p_ref, u1_ref, dinv_ref, b1_ref, w2_ref, u2_ref):
    dinv = dinv_ref[:, 0][:, None]
    h = dinv * (p_ref[0] + p_ref[1] + u1_ref[...]) + b1_ref[0][None, :]
    h = jnp.maximum(h, 0.0)
    t2 = jnp.dot(h, w2_ref[...], preferred_element_type=jnp.float32,
                 precision=lax.Precision.HIGHEST)
    u2_ref[...] = jnp.pad(t2 * dinv, ((0, 0), (0, t2.shape[1])))


def _fin_body(q_ref, u2_ref, dinv_ref, b2_ref, out_ref):
    o = out_ref.shape[1]
    dinv = dinv_ref[:, 0][:, None]
    out_ref[...] = (dinv * (q_ref[0][:, :o] + q_ref[1][:, :o] + u2_ref[:, :o])
                    + b2_ref[0][None, :])


def kernel(x0, x1, edge_index, Wp0, bp0, Wp1, bp1, W1, b1, W2, b2):
    n0, d0 = x0.shape
    n1, d1 = x1.shape
    n = n0 + n1
    h = W1.shape[0]
    o = W2.shape[1]
    e = edge_index.shape[1]
    f32 = jnp.float32

    n_pad = -(-(n + 1) // (NS * 8)) * (NS * 8)
    nch = -(-e // CHUNK)
    nch_tot = -(-nch // NW) * NW
    cpw = nch_tot // NW
    ep = nch_tot * CHUNK

    src = jnp.concatenate(
        [edge_index[0], jnp.zeros((ep - e,), edge_index.dtype)]).astype(jnp.int32)
    dst = jnp.concatenate(
        [edge_index[1], jnp.full((ep - e,), n, edge_index.dtype)]).astype(jnp.int32)
    srcs = src.reshape(nch_tot, CHUNK)
    dsts = dst.reshape(nch_tot, CHUNK)

    rps = n_pad // NS
    ones_deg = jnp.ones((CHUNK, DEG_W), f32)
    z_deg = jnp.zeros((rps, DEG_W), f32)
    z_h = jnp.zeros((rps, h), f32)

    degp = _make_deg_kernel(n_pad, cpw)(dsts, ones_deg, z_deg)

    x1p = jnp.pad(x1, ((0, 0), (0, d0 - d1)))
    xall = jnp.concatenate([x0, x1p], axis=0)
    wstk = jnp.stack([Wp0, jnp.pad(Wp1, ((0, d0 - d1), (0, 0)))])
    bstk = jnp.stack([bp0, bp1]).reshape(NC, 1, h)
    nb = n // BM
    bpt = n0 // BM
    u1, dinv8 = pl.pallas_call(
        _proj_body,
        grid=(nb,),
        in_specs=[
            pl.BlockSpec((BM, d0), lambda i: (i, 0)),
            pl.BlockSpec((1, d0, h), lambda i: (i // bpt, 0, 0)),
            pl.BlockSpec((1, 1, h), lambda i: (i // bpt, 0, 0)),
            pl.BlockSpec((h, h), lambda i: (0, 0)),
            pl.BlockSpec((NC, BM, DEG_W), lambda i: (0, i, 0)),
        ],
        out_specs=[
            pl.BlockSpec((BM, h), lambda i: (i, 0)),
            pl.BlockSpec((BM, DEG_W), lambda i: (i, 0)),
        ],
        out_shape=[
            jax.ShapeDtypeStruct((n, h), f32),
            jax.ShapeDtypeStruct((n, DEG_W), f32),
        ],
    )(xall, wstk, bstk, W1, degp)

    p1 = _make_agg_kernel(h, n_pad, cpw)(u1, srcs, dsts, z_h)

    u2 = pl.pallas_call(
        _mid_body,
        grid=(nb,),
        in_specs=[
            pl.BlockSpec((NC, BM, h), lambda i: (0, i, 0)),
            pl.BlockSpec((BM, h), lambda i: (i, 0)),
            pl.BlockSpec((BM, DEG_W), lambda i: (i, 0)),
            pl.BlockSpec((1, h), lambda i: (0, 0)),
            pl.BlockSpec((h, o), lambda i: (0, 0)),
        ],
        out_specs=pl.BlockSpec((BM, h), lambda i: (i, 0)),
        out_shape=jax.ShapeDtypeStruct((n, h), f32),
    )(p1, u1, dinv8, b1.reshape(1, h), W2)

    p2 = _make_agg_kernel(h, n_pad, cpw)(u2, srcs, dsts, z_h)

    out = pl.pallas_call(
        _fin_body,
        grid=(nb,),
        in_specs=[
            pl.BlockSpec((NC, BM, h), lambda i: (0, i, 0)),
            pl.BlockSpec((BM, h), lambda i: (i, 0)),
            pl.BlockSpec((BM, DEG_W), lambda i: (i, 0)),
            pl.BlockSpec((1, o), lambda i: (0, 0)),
        ],
        out_specs=pl.BlockSpec((BM, o), lambda i: (i, 0)),
        out_shape=jax.ShapeDtypeStruct((n, o), f32),
    )(p2, u2, dinv8, b2.reshape(1, o))
    return out
